# baseline (speedup 1.0000x reference)
import jax
import jax.numpy as jnp
from jax import lax
from jax.experimental import pallas as pl
from jax.experimental.pallas import tpu as pltpu
from jax.experimental.pallas import tpu_sc as plsc

NSEG = 10000
ROWS = 320000
D = 128
NC = 2
NS = 16
DH = D // NC
B = 400
RPW = ROWS // NS
NBLK = RPW // B
G = 16
NG = B // G
W = 512
ACC_ROWS = NSEG + W
ZROWS = ACC_ROWS // NS
WB = NSEG // 10


def _sc_body(feats_hbm, ids_hbm, zeros_hbm, out_hbm,
             f0, f1, i0, i1, sf0, sf1, si0, si1, win, win_ids, acc):
    c = lax.axis_index("c")
    s = lax.axis_index("s")
    zv = jnp.zeros((16,), jnp.float32)

    def start_read(blk, fbuf, ibuf, fsem, isem):
        row0 = (s * NBLK + blk) * B
        cf = pltpu.async_copy(
            feats_hbm.at[pl.ds(row0, B), pl.ds(c * DH, DH)], fbuf, fsem)
        ci = pltpu.async_copy(ids_hbm.at[pl.ds(row0, B)], ibuf, isem)
        return cf, ci

    def wait_read(fbuf, ibuf, fsem, isem):
        pltpu.make_async_copy(
            feats_hbm.at[pl.ds(0, B), pl.ds(0, DH)], fbuf, fsem).wait()
        pltpu.make_async_copy(ids_hbm.at[pl.ds(0, B)], ibuf, isem).wait()

    def zero_win():
        def zrow(i, carry):
            for j in range(4):
                win[i, pl.ds(16 * j, 16)] = zv
            return carry
        lax.fori_loop(0, W, zrow, 0)

    def build_win_ids(new_base):
        def idsrow(k, carry):
            win_ids[pl.ds(k * 16, 16)] = (
                new_base + k * 16 + lax.iota(jnp.int32, 16))
            return carry
        lax.fori_loop(0, W // 16, idsrow, 0)

    def flush(new_base):
        pltpu.sync_copy(win, acc.at[win_ids], add=True)
        zero_win()
        build_win_ids(new_base)

    def row_ops(fbuf, row, slot):
        for j in range(4):
            plsc.addupdate(win.at[slot, pl.ds(16 * j, 16)],
                           fbuf[row, pl.ds(16 * j, 16)])

    def consume(fbuf, ibuf, base):
        def group(g, base):
            idv = ibuf[pl.ds(g * G, G)]
            last = idv[G - 1]

            def fast(b):
                for r in range(G):
                    row_ops(fbuf, g * G + r, idv[r] - b)
                return b

            def slow(b):
                for r in range(G):
                    idr = idv[r]

                    def do_flush(_):
                        flush(idr)
                        return idr

                    b = lax.cond(idr - b >= W, do_flush, lambda x: x, b)
                    row_ops(fbuf, g * G + r, idr - b)
                return b

            return lax.cond(last - base < W, fast, slow, base)

        return lax.fori_loop(0, NG, group, base)

    start_read(0, f0, i0, sf0, si0)
    start_read(1, f1, i1, sf1, si1)

    pltpu.sync_copy(zeros_hbm, acc.at[pl.ds(s * ZROWS, ZROWS)])
    zero_win()
    build_win_ids(0)
    plsc.subcore_barrier()

    def pair(p, base):
        blk0 = 2 * p
        wait_read(f0, i0, sf0, si0)
        base = consume(f0, i0, base)

        @pl.when(blk0 + 2 < NBLK)
        def _():
            start_read(blk0 + 2, f0, i0, sf0, si0)

        wait_read(f1, i1, sf1, si1)
        base = consume(f1, i1, base)

        @pl.when(blk0 + 3 < NBLK)
        def _():
            start_read(blk0 + 3, f1, i1, sf1, si1)

        return base

    lax.fori_loop(0, NBLK // 2, pair, jnp.int32(0))

    pltpu.sync_copy(win, acc.at[win_ids], add=True)
    plsc.subcore_barrier()

    @pl.when(s < 10)
    def _():
        pltpu.sync_copy(
            acc.at[pl.ds(s * WB, WB)],
            out_hbm.at[pl.ds(s * WB, WB), pl.ds(c * DH, DH)],
        )


@jax.jit
def _pool_sum(feats, ids, zeros):
    mesh = plsc.VectorSubcoreMesh(
        core_axis_name="c", subcore_axis_name="s", num_cores=NC, num_subcores=NS
    )
    return pl.kernel(
        _sc_body,
        out_type=jax.ShapeDtypeStruct((NSEG, D), jnp.float32),
        mesh=mesh,
        scratch_types=[
            pltpu.VMEM((B, DH), jnp.float32),
            pltpu.VMEM((B, DH), jnp.float32),
            pltpu.VMEM((B,), jnp.int32),
            pltpu.VMEM((B,), jnp.int32),
            pltpu.SemaphoreType.DMA,
            pltpu.SemaphoreType.DMA,
            pltpu.SemaphoreType.DMA,
            pltpu.SemaphoreType.DMA,
            pltpu.VMEM((W, DH), jnp.float32),
            pltpu.VMEM((W,), jnp.int32),
            pltpu.VMEM_SHARED((ACC_ROWS, DH), jnp.float32),
        ],
        compiler_params=pltpu.CompilerParams(use_tc_tiling_on_sc=False),
    )(feats, ids, zeros)


def kernel(feats, batch):
    ids = batch.astype(jnp.int32)
    zeros = jnp.zeros((ZROWS, DH), jnp.float32)
    return _pool_sum(feats, ids, zeros)


# speedup vs baseline: 2.6754x; 2.6754x over previous
import jax
import jax.numpy as jnp
from jax import lax
from jax.experimental import pallas as pl
from jax.experimental.pallas import tpu as pltpu
from jax.experimental.pallas import tpu_sc as plsc

NSEG = 10000
ROWS = 320000
D = 128
NC = 2
NS = 16
DH = D // NC
B = 200
RPW = ROWS // NS
NBLK = RPW // B
NBUF = 7
RDA = 2
ZROWS = NSEG // NS
WB = NSEG // 10


def _sc_body(feats_hbm, ids_hbm, zeros_hbm, out_hbm, *scratch):
    feats_bufs = scratch[0:NBUF]
    ids_bufs = scratch[NBUF:2 * NBUF]
    sems_f = scratch[2 * NBUF:3 * NBUF]
    sems_i = scratch[3 * NBUF:4 * NBUF]
    sems_s = scratch[4 * NBUF:5 * NBUF]
    acc = scratch[5 * NBUF]

    c = lax.axis_index("c")
    s = lax.axis_index("s")

    def start_read(b, slot):
        gb = s * NBLK + b
        row0 = gb * B
        cf = pltpu.async_copy(
            feats_hbm.at[pl.ds(row0, B), pl.ds(c * DH, DH)],
            feats_bufs[slot], sems_f[slot])
        ci = pltpu.async_copy(ids_hbm.at[gb], ids_bufs[slot], sems_i[slot])
        return cf, ci

    reads = {}
    for p in range(RDA):
        reads[p] = start_read(p, p % NBUF)

    pltpu.sync_copy(zeros_hbm, acc.at[pl.ds(s * ZROWS, ZROWS)])
    plsc.subcore_barrier()

    scats = {}
    for b in range(NBLK):
        slot = b % NBUF
        cf, ci = reads.pop(b)
        cf.wait()
        ci.wait()
        scats[b] = pltpu.async_copy(
            feats_bufs[slot], acc.at[ids_bufs[slot]], sems_s[slot], add=True)
        nb = b + RDA
        if nb < NBLK:
            nslot = nb % NBUF
            prev = nb - NBUF
            if prev >= 0:
                scats.pop(prev).wait()
            reads[nb] = start_read(nb, nslot)
    for b in sorted(scats):
        scats[b].wait()
    plsc.subcore_barrier()

    @pl.when(s < 10)
    def _():
        pltpu.sync_copy(
            acc.at[pl.ds(s * WB, WB)],
            out_hbm.at[pl.ds(s * WB, WB), pl.ds(c * DH, DH)],
        )


@jax.jit
def _pool_sum(feats, ids3, zeros):
    mesh = plsc.VectorSubcoreMesh(
        core_axis_name="c", subcore_axis_name="s", num_cores=NC, num_subcores=NS
    )
    return pl.kernel(
        _sc_body,
        out_type=jax.ShapeDtypeStruct((NSEG, D), jnp.float32),
        mesh=mesh,
        scratch_types=(
            [pltpu.VMEM((B, DH), jnp.float32) for _ in range(NBUF)]
            + [pltpu.VMEM((B,), jnp.int32) for _ in range(NBUF)]
            + [pltpu.SemaphoreType.DMA for _ in range(3 * NBUF)]
            + [pltpu.VMEM_SHARED((NSEG, DH), jnp.float32)]
        ),
        compiler_params=pltpu.CompilerParams(use_tc_tiling_on_sc=False),
    )(feats, ids3, zeros)


def kernel(feats, batch):
    ids3 = batch.astype(jnp.int32).reshape(ROWS // B, B)
    zeros = jnp.zeros((ZROWS, DH), jnp.float32)
    return _pool_sum(feats, ids3, zeros)


# speedup vs baseline: 2.7766x; 1.0378x over previous
import jax
import jax.numpy as jnp
from jax import lax
from jax.experimental import pallas as pl
from jax.experimental.pallas import tpu as pltpu
from jax.experimental.pallas import tpu_sc as plsc

NSEG = 10000
ROWS = 320000
D = 128
NC = 2
NS = 16
DH = D // NC
B = 200
RPW = ROWS // NS
NBLK = RPW // B
NBUF = 7
RDA = 3
ZROWS = NSEG // NS
WB = NSEG // 10


def _sc_body(feats_hbm, ids_hbm, zeros_hbm, out_hbm, *scratch):
    feats_bufs = scratch[0:NBUF]
    ids_bufs = scratch[NBUF:2 * NBUF]
    sems_f = scratch[2 * NBUF:3 * NBUF]
    sems_i = scratch[3 * NBUF:4 * NBUF]
    sems_s = scratch[4 * NBUF:5 * NBUF]
    acc = scratch[5 * NBUF]

    c = lax.axis_index("c")
    s = lax.axis_index("s")

    def start_read(b, slot):
        gb = s * NBLK + b
        row0 = gb * B
        cf = pltpu.async_copy(
            feats_hbm.at[pl.ds(row0, B), pl.ds(c * DH, DH)],
            feats_bufs[slot], sems_f[slot])
        ci = pltpu.async_copy(ids_hbm.at[gb], ids_bufs[slot], sems_i[slot])
        return cf, ci

    reads = {}
    for p in range(RDA):
        reads[p] = start_read(p, p % NBUF)

    pltpu.sync_copy(zeros_hbm, acc.at[pl.ds(s * ZROWS, ZROWS)])
    plsc.subcore_barrier()

    scats = {}
    for b in range(NBLK):
        slot = b % NBUF
        cf, ci = reads.pop(b)
        cf.wait()
        ci.wait()
        scats[b] = pltpu.async_copy(
            feats_bufs[slot], acc.at[ids_bufs[slot]], sems_s[slot], add=True)
        nb = b + RDA
        if nb < NBLK:
            nslot = nb % NBUF
            prev = nb - NBUF
            if prev >= 0:
                scats.pop(prev).wait()
            reads[nb] = start_read(nb, nslot)
    for b in sorted(scats):
        scats[b].wait()
    plsc.subcore_barrier()

    @pl.when(s < 10)
    def _():
        pltpu.sync_copy(
            acc.at[pl.ds(s * WB, WB)],
            out_hbm.at[pl.ds(s * WB, WB), pl.ds(c * DH, DH)],
        )


@jax.jit
def _pool_sum(feats, ids3, zeros):
    mesh = plsc.VectorSubcoreMesh(
        core_axis_name="c", subcore_axis_name="s", num_cores=NC, num_subcores=NS
    )
    return pl.kernel(
        _sc_body,
        out_type=jax.ShapeDtypeStruct((NSEG, D), jnp.float32),
        mesh=mesh,
        scratch_types=(
            [pltpu.VMEM((B, DH), jnp.float32) for _ in range(NBUF)]
            + [pltpu.VMEM((B,), jnp.int32) for _ in range(NBUF)]
            + [pltpu.SemaphoreType.DMA for _ in range(3 * NBUF)]
            + [pltpu.VMEM_SHARED((NSEG, DH), jnp.float32)]
        ),
        compiler_params=pltpu.CompilerParams(use_tc_tiling_on_sc=False),
    )(feats, ids3, zeros)


def kernel(feats, batch):
    ids3 = batch.astype(jnp.int32).reshape(ROWS // B, B)
    zeros = jnp.zeros((ZROWS, DH), jnp.float32)
    return _pool_sum(feats, ids3, zeros)


# speedup vs baseline: 2.8295x; 1.0191x over previous
import jax
import jax.numpy as jnp
from jax import lax
from jax.experimental import pallas as pl
from jax.experimental.pallas import tpu as pltpu
from jax.experimental.pallas import tpu_sc as plsc

NSEG = 10000
ROWS = 320000
D = 128
NC = 2
NS = 16
DH = D // NC
B = 200
RPW = ROWS // NS
NBLK = RPW // B
NBUF = 7
RDA = 4
ZROWS = NSEG // NS
WB = NSEG // 10


def _sc_body(feats_hbm, ids_hbm, zeros_hbm, out_hbm, *scratch):
    feats_bufs = scratch[0:NBUF]
    ids_bufs = scratch[NBUF:2 * NBUF]
    sems_f = scratch[2 * NBUF:3 * NBUF]
    sems_i = scratch[3 * NBUF:4 * NBUF]
    sems_s = scratch[4 * NBUF:5 * NBUF]
    acc = scratch[5 * NBUF]

    c = lax.axis_index("c")
    s = lax.axis_index("s")

    def start_read(b, slot):
        gb = s * NBLK + b
        row0 = gb * B
        cf = pltpu.async_copy(
            feats_hbm.at[pl.ds(row0, B), pl.ds(c * DH, DH)],
            feats_bufs[slot], sems_f[slot])
        ci = pltpu.async_copy(ids_hbm.at[gb], ids_bufs[slot], sems_i[slot])
        return cf, ci

    reads = {}
    for p in range(RDA):
        reads[p] = start_read(p, p % NBUF)

    pltpu.sync_copy(zeros_hbm, acc.at[pl.ds(s * ZROWS, ZROWS)])
    plsc.subcore_barrier()

    scats = {}
    for b in range(NBLK):
        slot = b % NBUF
        cf, ci = reads.pop(b)
        cf.wait()
        ci.wait()
        scats[b] = pltpu.async_copy(
            feats_bufs[slot], acc.at[ids_bufs[slot]], sems_s[slot], add=True)
        nb = b + RDA
        if nb < NBLK:
            nslot = nb % NBUF
            prev = nb - NBUF
            if prev >= 0:
                scats.pop(prev).wait()
            reads[nb] = start_read(nb, nslot)
    for b in sorted(scats):
        scats[b].wait()
    plsc.subcore_barrier()

    @pl.when(s < 10)
    def _():
        pltpu.sync_copy(
            acc.at[pl.ds(s * WB, WB)],
            out_hbm.at[pl.ds(s * WB, WB), pl.ds(c * DH, DH)],
        )


@jax.jit
def _pool_sum(feats, ids3, zeros):
    mesh = plsc.VectorSubcoreMesh(
        core_axis_name="c", subcore_axis_name="s", num_cores=NC, num_subcores=NS
    )
    return pl.kernel(
        _sc_body,
        out_type=jax.ShapeDtypeStruct((NSEG, D), jnp.float32),
        mesh=mesh,
        scratch_types=(
            [pltpu.VMEM((B, DH), jnp.float32) for _ in range(NBUF)]
            + [pltpu.VMEM((B,), jnp.int32) for _ in range(NBUF)]
            + [pltpu.SemaphoreType.DMA for _ in range(3 * NBUF)]
            + [pltpu.VMEM_SHARED((NSEG, DH), jnp.float32)]
        ),
        compiler_params=pltpu.CompilerParams(use_tc_tiling_on_sc=False),
    )(feats, ids3, zeros)


def kernel(feats, batch):
    ids3 = batch.astype(jnp.int32).reshape(ROWS // B, B)
    zeros = jnp.zeros((ZROWS, DH), jnp.float32)
    return _pool_sum(feats, ids3, zeros)


# speedup vs baseline: 2.8558x; 1.0093x over previous
import jax
import jax.numpy as jnp
from jax import lax
from jax.experimental import pallas as pl
from jax.experimental.pallas import tpu as pltpu
from jax.experimental.pallas import tpu_sc as plsc

NSEG = 10000
ROWS = 320000
D = 128
NC = 2
NS = 16
DH = D // NC
B = 200
RPW = ROWS // NS
NBLK = RPW // B
NBUF = 7
RDA = 5
ZROWS = NSEG // NS
WB = NSEG // 10


def _sc_body(feats_hbm, ids_hbm, zeros_hbm, out_hbm, *scratch):
    feats_bufs = scratch[0:NBUF]
    ids_bufs = scratch[NBUF:2 * NBUF]
    sems_f = scratch[2 * NBUF:3 * NBUF]
    sems_i = scratch[3 * NBUF:4 * NBUF]
    sems_s = scratch[4 * NBUF:5 * NBUF]
    acc = scratch[5 * NBUF]

    c = lax.axis_index("c")
    s = lax.axis_index("s")

    def start_read(b, slot):
        gb = s * NBLK + b
        row0 = gb * B
        cf = pltpu.async_copy(
            feats_hbm.at[pl.ds(row0, B), pl.ds(c * DH, DH)],
            feats_bufs[slot], sems_f[slot])
        ci = pltpu.async_copy(ids_hbm.at[gb], ids_bufs[slot], sems_i[slot])
        return cf, ci

    reads = {}
    for p in range(RDA):
        reads[p] = start_read(p, p % NBUF)

    pltpu.sync_copy(zeros_hbm, acc.at[pl.ds(s * ZROWS, ZROWS)])
    plsc.subcore_barrier()

    scats = {}
    for b in range(NBLK):
        slot = b % NBUF
        cf, ci = reads.pop(b)
        cf.wait()
        ci.wait()
        scats[b] = pltpu.async_copy(
            feats_bufs[slot], acc.at[ids_bufs[slot]], sems_s[slot], add=True)
        nb = b + RDA
        if nb < NBLK:
            nslot = nb % NBUF
            prev = nb - NBUF
            if prev >= 0:
                scats.pop(prev).wait()
            reads[nb] = start_read(nb, nslot)
    for b in sorted(scats):
        scats[b].wait()
    plsc.subcore_barrier()

    @pl.when(s < 10)
    def _():
        pltpu.sync_copy(
            acc.at[pl.ds(s * WB, WB)],
            out_hbm.at[pl.ds(s * WB, WB), pl.ds(c * DH, DH)],
        )


@jax.jit
def _pool_sum(feats, ids3, zeros):
    mesh = plsc.VectorSubcoreMesh(
        core_axis_name="c", subcore_axis_name="s", num_cores=NC, num_subcores=NS
    )
    return pl.kernel(
        _sc_body,
        out_type=jax.ShapeDtypeStruct((NSEG, D), jnp.float32),
        mesh=mesh,
        scratch_types=(
            [pltpu.VMEM((B, DH), jnp.float32) for _ in range(NBUF)]
            + [pltpu.VMEM((B,), jnp.int32) for _ in range(NBUF)]
            + [pltpu.SemaphoreType.DMA for _ in range(3 * NBUF)]
            + [pltpu.VMEM_SHARED((NSEG, DH), jnp.float32)]
        ),
        compiler_params=pltpu.CompilerParams(use_tc_tiling_on_sc=False),
    )(feats, ids3, zeros)


def kernel(feats, batch):
    ids3 = batch.astype(jnp.int32).reshape(ROWS // B, B)
    zeros = jnp.zeros((ZROWS, DH), jnp.float32)
    return _pool_sum(feats, ids3, zeros)
